# baseline (speedup 1.0000x reference)
import jax
import jax.numpy as jnp
from jax.experimental import pallas as pl
from jax.experimental.pallas import tpu as pltpu

_ALPHA = 4
_IDX = (0, 4, 8, 12, 16, 21, 25, 29, 33, 37, 42, 46, 50, 54, 58, 63)
_C, _T, _H, _W = 3, 64, 224, 224
_TS = _T // _ALPHA
_CH = 4
_WPC = _T // _CH
_NCH = _C * _WPC
_SELC = _TS // _WPC
_K = 16
_D = 8


def _body(x_ref, slow_ref, fast_ref, *scratch):
    bufs = scratch[:_K]
    sin = scratch[_K:2 * _K]
    sout = scratch[2 * _K:3 * _K]

    def in_cp(m):
        c, w = divmod(m, _WPC)
        return pltpu.make_async_copy(
            x_ref.at[c, pl.ds(w * _CH, _CH)], bufs[m % _K], sin[m % _K]
        )

    def out_cps(m):
        c, w = divmod(m, _WPC)
        cps = [
            pltpu.make_async_copy(
                bufs[m % _K], fast_ref.at[c, pl.ds(w * _CH, _CH)], sout[m % _K]
            )
        ]
        for k in range(_SELC):
            j = w * _SELC + k
            g = _IDX[j] - w * _CH
            cps.append(
                pltpu.make_async_copy(
                    bufs[m % _K].at[g], slow_ref.at[c, j], sout[m % _K]
                )
            )
        return cps

    pending = {}
    for m in range(_D):
        in_cp(m).start()
    for m in range(_NCH):
        in_cp(m).wait()
        cps = out_cps(m)
        for cp in cps:
            cp.start()
        pending[m] = cps
        nm = m + _D
        if nm < _NCH:
            prev = nm - _K
            if prev >= 0:
                for cp in pending.pop(prev):
                    cp.wait()
            in_cp(nm).start()
    for m in sorted(pending):
        for cp in pending[m]:
            cp.wait()


def kernel(frames):
    C, T, H, W = frames.shape
    Ts = T // _ALPHA
    slow, fast = pl.pallas_call(
        _body,
        in_specs=[pl.BlockSpec(memory_space=pl.ANY)],
        out_specs=[
            pl.BlockSpec(memory_space=pl.ANY),
            pl.BlockSpec(memory_space=pl.ANY),
        ],
        out_shape=[
            jax.ShapeDtypeStruct((C, Ts, H, W), frames.dtype),
            jax.ShapeDtypeStruct((C, T, H, W), frames.dtype),
        ],
        scratch_shapes=(
            [pltpu.VMEM((_CH, _H, _W), jnp.float32)] * _K
            + [pltpu.SemaphoreType.DMA] * (2 * _K)
        ),
    )(frames)
    return (slow, fast)


# speedup vs baseline: 1.0212x; 1.0212x over previous
import jax
import jax.numpy as jnp
from jax.experimental import pallas as pl
from jax.experimental.pallas import tpu as pltpu

_ALPHA = 4
_IDX = (0, 4, 8, 12, 16, 21, 25, 29, 33, 37, 42, 46, 50, 54, 58, 63)
_C, _T, _H, _W = 3, 64, 224, 224
_TS = _T // _ALPHA
_CH = 16
_WPC = _T // _CH
_NCH = _C * _WPC
_SELC = _TS // _WPC
_K = 6
_D = 3


def _body(x_ref, slow_ref, fast_ref, *scratch):
    bufs = scratch[:_K]
    sin = scratch[_K:2 * _K]
    sout = scratch[2 * _K:3 * _K]

    def in_cp(m):
        c, w = divmod(m, _WPC)
        return pltpu.make_async_copy(
            x_ref.at[c, pl.ds(w * _CH, _CH)], bufs[m % _K], sin[m % _K]
        )

    def out_cps(m):
        c, w = divmod(m, _WPC)
        cps = [
            pltpu.make_async_copy(
                bufs[m % _K], fast_ref.at[c, pl.ds(w * _CH, _CH)], sout[m % _K]
            )
        ]
        for k in range(_SELC):
            j = w * _SELC + k
            g = _IDX[j] - w * _CH
            cps.append(
                pltpu.make_async_copy(
                    bufs[m % _K].at[g], slow_ref.at[c, j], sout[m % _K]
                )
            )
        return cps

    pending = {}
    for m in range(_D):
        in_cp(m).start()
    for m in range(_NCH):
        in_cp(m).wait()
        cps = out_cps(m)
        for cp in cps:
            cp.start()
        pending[m] = cps
        nm = m + _D
        if nm < _NCH:
            prev = nm - _K
            if prev >= 0:
                for cp in pending.pop(prev):
                    cp.wait()
            in_cp(nm).start()
    for m in sorted(pending):
        for cp in pending[m]:
            cp.wait()


def kernel(frames):
    C, T, H, W = frames.shape
    Ts = T // _ALPHA
    slow, fast = pl.pallas_call(
        _body,
        in_specs=[pl.BlockSpec(memory_space=pl.ANY)],
        out_specs=[
            pl.BlockSpec(memory_space=pl.ANY),
            pl.BlockSpec(memory_space=pl.ANY),
        ],
        out_shape=[
            jax.ShapeDtypeStruct((C, Ts, H, W), frames.dtype),
            jax.ShapeDtypeStruct((C, T, H, W), frames.dtype),
        ],
        scratch_shapes=(
            [pltpu.VMEM((_CH, _H, _W), jnp.float32)] * _K
            + [pltpu.SemaphoreType.DMA] * (2 * _K)
        ),
    )(frames)
    return (slow, fast)
